# Initial kernel scaffold; baseline (speedup 1.0000x reference)
import jax
import jax.numpy as jnp
from jax.experimental import pallas as pl


def kernel(x, edge_index, edge_attr, mlp_w1, mlp_w2, root, bias):
    raise NotImplementedError("write your pallas kernel here")


# speedup vs baseline: 3.2270x; 3.2270x over previous
import functools

import jax
import jax.numpy as jnp
from jax import lax
from jax.experimental import pallas as pl
from jax.experimental.pallas import tpu as pltpu
from jax.experimental.pallas import tpu_sc as plsc

NC, NS = 2, 16
NW = NC * NS
CH = 128


def _gather_body(nch, x_hbm, idx_hbm, out_hbm, idx_v, rows_v, sem):
    c = lax.axis_index("c")
    s = lax.axis_index("s")
    wid = s * NC + c
    epw = nch * CH
    pltpu.sync_copy(idx_hbm.at[wid], idx_v)

    def chunk(j, carry):
        pltpu.async_copy(
            x_hbm.at[idx_v.at[j]],
            rows_v.at[pl.ds(j * CH, CH)],
            sem,
        ).wait()
        return carry

    lax.fori_loop(0, nch, chunk, 0)
    pltpu.sync_copy(rows_v, out_hbm.at[pl.ds(wid * epw, epw)])


def _sc_gather(x, idx_grouped, nin, nch):
    epw = nch * CH
    run = pl.kernel(
        functools.partial(_gather_body, nch),
        out_type=jax.ShapeDtypeStruct((NW * epw, nin), jnp.float32),
        mesh=plsc.VectorSubcoreMesh(core_axis_name="c", subcore_axis_name="s"),
        scratch_types=[
            pltpu.VMEM((nch, CH), jnp.int32),
            pltpu.VMEM((epw, nin), jnp.float32),
            pltpu.SemaphoreType.DMA,
        ],
        compiler_params=pltpu.CompilerParams(use_tc_tiling_on_sc=False),
    )
    return run(x, idx_grouped)


def _scatter_body(nch, n_acc, n_out, nout, msg_hbm, idx_hbm, zeros_hbm,
                  part_hbm, idx_v, rows_v, tmp_v, acc_sh, sem):
    c = lax.axis_index("c")
    s = lax.axis_index("s")
    wid = s * NC + c
    epw = nch * CH
    cnt = n_acc // NS

    pltpu.sync_copy(zeros_hbm.at[pl.ds(s * cnt, cnt)], tmp_v)
    pltpu.sync_copy(tmp_v, acc_sh.at[pl.ds(s * cnt, cnt)])

    pltpu.sync_copy(idx_hbm.at[wid], idx_v)
    pltpu.sync_copy(msg_hbm.at[pl.ds(wid * epw, epw)], rows_v)
    plsc.subcore_barrier()

    def chunk(j, carry):
        pltpu.sync_copy(
            rows_v.at[pl.ds(j * CH, CH)],
            acc_sh.at[idx_v.at[j]],
            add=True,
        )
        return carry

    lax.fori_loop(0, nch, chunk, 0)
    plsc.subcore_barrier()

    ocnt = n_out // NS
    pltpu.sync_copy(acc_sh.at[pl.ds(s * ocnt, ocnt)], tmp_v.at[pl.ds(0, ocnt)])
    pltpu.sync_copy(tmp_v.at[pl.ds(0, ocnt)],
                    part_hbm.at[c].at[pl.ds(s * ocnt, ocnt)])


def _sc_scatter(msg, idx_grouped, zeros_acc, n_acc, n_out, nout, nch):
    epw = nch * CH
    run = pl.kernel(
        functools.partial(_scatter_body, nch, n_acc, n_out, nout),
        out_type=jax.ShapeDtypeStruct((NC, n_out, nout), jnp.float32),
        mesh=plsc.VectorSubcoreMesh(core_axis_name="c", subcore_axis_name="s"),
        scratch_types=[
            pltpu.VMEM((nch, CH), jnp.int32),
            pltpu.VMEM((epw, nout), jnp.float32),
            pltpu.VMEM((n_acc // NS, nout), jnp.float32),
            pltpu.VMEM_SHARED((n_acc, nout), jnp.float32),
            pltpu.SemaphoreType.DMA,
        ],
        compiler_params=pltpu.CompilerParams(use_tc_tiling_on_sc=False),
    )
    return run(msg, idx_grouped, zeros_acc)


def _edge_tc(ea_ref, xj_ref, w1_ref, w2_ref, rep_ref, sel_ref, msg_ref):
    a = jnp.dot(ea_ref[...], w1_ref[...], preferred_element_type=jnp.float32)
    r = jnp.maximum(a, 0.0)
    h = jnp.dot(r, w2_ref[...], preferred_element_type=jnp.float32)
    xr = jnp.dot(xj_ref[...], rep_ref[...], preferred_element_type=jnp.float32)
    msg_ref[...] = jnp.dot(xr * h, sel_ref[...],
                           preferred_element_type=jnp.float32)


def _tc_edge(ea, xj, w1, w2, rep, sel, be, interpret=False):
    e_pad, nin = ea.shape
    hid = w2.shape[1]
    nout = sel.shape[1]
    grid = e_pad // be
    return pl.pallas_call(
        _edge_tc,
        grid=(grid,),
        in_specs=[
            pl.BlockSpec((be, nin), lambda i: (i, 0)),
            pl.BlockSpec((be, nin), lambda i: (i, 0)),
            pl.BlockSpec((nin, hid), lambda i: (0, 0)),
            pl.BlockSpec((hid, hid), lambda i: (0, 0)),
            pl.BlockSpec((nin, hid), lambda i: (0, 0)),
            pl.BlockSpec((hid, nout), lambda i: (0, 0)),
        ],
        out_specs=pl.BlockSpec((be, nout), lambda i: (i, 0)),
        out_shape=jax.ShapeDtypeStruct((e_pad, nout), jnp.float32),
        interpret=interpret,
    )(ea, xj, w1, w2, rep, sel)


def _combine_tc(p_ref, x_ref, root_ref, bias_ref, out_ref):
    xr = jnp.dot(x_ref[...], root_ref[...], preferred_element_type=jnp.float32)
    out_ref[...] = p_ref[0] + p_ref[1] + xr + bias_ref[...]


def _tc_combine(parts, x, root, bias2d, bn, interpret=False):
    n, nin = x.shape
    nout = root.shape[1]
    grid = n // bn
    return pl.pallas_call(
        _combine_tc,
        grid=(grid,),
        in_specs=[
            pl.BlockSpec((NC, bn, nout), lambda i: (0, i, 0)),
            pl.BlockSpec((bn, nin), lambda i: (i, 0)),
            pl.BlockSpec((nin, nout), lambda i: (0, 0)),
            pl.BlockSpec((1, nout), lambda i: (0, 0)),
        ],
        out_specs=pl.BlockSpec((bn, nout), lambda i: (i, 0)),
        out_shape=jax.ShapeDtypeStruct((n, nout), jnp.float32),
        interpret=interpret,
    )(parts, x, root, bias2d)


def kernel(x, edge_index, edge_attr, mlp_w1, mlp_w2, root, bias):
    n, nin = x.shape
    e = edge_index.shape[1]
    hid = mlp_w1.shape[1]
    nout = root.shape[1]
    assert n % NS == 0

    nch = -(-e // (NW * CH))
    e_pad = NW * nch * CH
    src = edge_index[0]
    dst = edge_index[1]
    pad = e_pad - e
    if pad:
        src = jnp.concatenate([src, jnp.zeros((pad,), jnp.int32)])
        dst = jnp.concatenate([dst, jnp.full((pad,), n, jnp.int32)])
        ea = jnp.concatenate([edge_attr,
                              jnp.zeros((pad, nin), edge_attr.dtype)])
    else:
        ea = edge_attr
    src_g = src.reshape(NW, nch, CH)
    dst_g = dst.reshape(NW, nch, CH)

    n_acc = -(-(n + 1) // NS) * NS
    zeros_acc = jnp.zeros((n_acc, nout), jnp.float32)

    ii = lax.broadcasted_iota(jnp.int32, (nin, hid), 0)
    cc = lax.broadcasted_iota(jnp.int32, (nin, hid), 1)
    rep = (cc // nout == ii).astype(jnp.float32)
    c2 = lax.broadcasted_iota(jnp.int32, (hid, nout), 0)
    oo = lax.broadcasted_iota(jnp.int32, (hid, nout), 1)
    sel = (c2 % nout == oo).astype(jnp.float32)

    xj = _sc_gather(x, src_g, nin, nch)
    msg = _tc_edge(ea, xj, mlp_w1, mlp_w2, rep, sel, be=2048)
    parts = _sc_scatter(msg, dst_g, zeros_acc, n_acc, n, nout, nch)
    out = _tc_combine(parts, x, root, bias.reshape(1, nout), bn=2000)
    return out


# speedup vs baseline: 3.3200x; 1.0288x over previous
import functools

import jax
import jax.numpy as jnp
from jax import lax
from jax.experimental import pallas as pl
from jax.experimental.pallas import tpu as pltpu
from jax.experimental.pallas import tpu_sc as plsc

NC, NS = 2, 16
NW = NC * NS
CH = 128


def _gather_body(nch, x_hbm, idx_hbm, out_hbm, idx_v, rows_v, sem):
    c = lax.axis_index("c")
    s = lax.axis_index("s")
    wid = s * NC + c
    epw = nch * CH
    pltpu.sync_copy(idx_hbm.at[wid], idx_v)

    def fire(j, carry):
        pltpu.make_async_copy(
            x_hbm.at[idx_v.at[j]],
            rows_v.at[pl.ds(j * CH, CH)],
            sem,
        ).start()
        return carry

    def drain(j, carry):
        pltpu.make_async_copy(
            x_hbm.at[idx_v.at[0]],
            rows_v.at[pl.ds(0, CH)],
            sem,
        ).wait()
        return carry

    lax.fori_loop(0, nch, fire, 0)
    lax.fori_loop(0, nch, drain, 0)
    pltpu.sync_copy(rows_v, out_hbm.at[pl.ds(wid * epw, epw)])


def _sc_gather(x, idx_grouped, nin, nch):
    epw = nch * CH
    run = pl.kernel(
        functools.partial(_gather_body, nch),
        out_type=jax.ShapeDtypeStruct((NW * epw, nin), jnp.float32),
        mesh=plsc.VectorSubcoreMesh(core_axis_name="c", subcore_axis_name="s"),
        scratch_types=[
            pltpu.VMEM((nch, CH), jnp.int32),
            pltpu.VMEM((epw, nin), jnp.float32),
            pltpu.SemaphoreType.DMA,
        ],
        compiler_params=pltpu.CompilerParams(use_tc_tiling_on_sc=False),
    )
    return run(x, idx_grouped)


def _scatter_body(nch, n_acc, n_out, nout, msg_hbm, idx_hbm, zeros_hbm,
                  part_hbm, idx_v, rows_v, tmp_v, acc_sh, sem):
    c = lax.axis_index("c")
    s = lax.axis_index("s")
    wid = s * NC + c
    epw = nch * CH
    cnt = n_acc // NS

    pltpu.sync_copy(zeros_hbm.at[pl.ds(s * cnt, cnt)], tmp_v)
    pltpu.sync_copy(tmp_v, acc_sh.at[pl.ds(s * cnt, cnt)])

    pltpu.sync_copy(idx_hbm.at[wid], idx_v)
    pltpu.sync_copy(msg_hbm.at[pl.ds(wid * epw, epw)], rows_v)
    plsc.subcore_barrier()

    def fire(j, carry):
        pltpu.async_copy(
            rows_v.at[pl.ds(j * CH, CH)],
            acc_sh.at[idx_v.at[j]],
            sem,
            add=True,
        )
        return carry

    def drain(j, carry):
        pltpu.make_async_copy(
            rows_v.at[pl.ds(0, CH)],
            acc_sh.at[idx_v.at[0]],
            sem,
        ).wait()
        return carry

    lax.fori_loop(0, nch, fire, 0)
    lax.fori_loop(0, nch, drain, 0)
    plsc.subcore_barrier()

    ocnt = n_out // NS
    pltpu.sync_copy(acc_sh.at[pl.ds(s * ocnt, ocnt)], tmp_v.at[pl.ds(0, ocnt)])
    pltpu.sync_copy(tmp_v.at[pl.ds(0, ocnt)],
                    part_hbm.at[c].at[pl.ds(s * ocnt, ocnt)])


def _sc_scatter(msg, idx_grouped, zeros_acc, n_acc, n_out, nout, nch):
    epw = nch * CH
    run = pl.kernel(
        functools.partial(_scatter_body, nch, n_acc, n_out, nout),
        out_type=jax.ShapeDtypeStruct((NC, n_out, nout), jnp.float32),
        mesh=plsc.VectorSubcoreMesh(core_axis_name="c", subcore_axis_name="s"),
        scratch_types=[
            pltpu.VMEM((nch, CH), jnp.int32),
            pltpu.VMEM((epw, nout), jnp.float32),
            pltpu.VMEM((n_acc // NS, nout), jnp.float32),
            pltpu.VMEM_SHARED((n_acc, nout), jnp.float32),
            pltpu.SemaphoreType.DMA,
        ],
        compiler_params=pltpu.CompilerParams(use_tc_tiling_on_sc=False),
    )
    return run(msg, idx_grouped, zeros_acc)


def _edge_tc(ea_ref, xj_ref, w1_ref, w2_ref, rep_ref, sel_ref, msg_ref):
    a = jnp.dot(ea_ref[...], w1_ref[...], preferred_element_type=jnp.float32)
    r = jnp.maximum(a, 0.0)
    h = jnp.dot(r, w2_ref[...], preferred_element_type=jnp.float32)
    xr = jnp.dot(xj_ref[...], rep_ref[...], preferred_element_type=jnp.float32)
    msg_ref[...] = jnp.dot(xr * h, sel_ref[...],
                           preferred_element_type=jnp.float32)


def _tc_edge(ea, xj, w1, w2, rep, sel, be, interpret=False):
    e_pad, nin = ea.shape
    hid = w2.shape[1]
    nout = sel.shape[1]
    grid = e_pad // be
    return pl.pallas_call(
        _edge_tc,
        grid=(grid,),
        in_specs=[
            pl.BlockSpec((be, nin), lambda i: (i, 0)),
            pl.BlockSpec((be, nin), lambda i: (i, 0)),
            pl.BlockSpec((nin, hid), lambda i: (0, 0)),
            pl.BlockSpec((hid, hid), lambda i: (0, 0)),
            pl.BlockSpec((nin, hid), lambda i: (0, 0)),
            pl.BlockSpec((hid, nout), lambda i: (0, 0)),
        ],
        out_specs=pl.BlockSpec((be, nout), lambda i: (i, 0)),
        out_shape=jax.ShapeDtypeStruct((e_pad, nout), jnp.float32),
        interpret=interpret,
    )(ea, xj, w1, w2, rep, sel)


def _combine_tc(p_ref, x_ref, root_ref, bias_ref, out_ref):
    xr = jnp.dot(x_ref[...], root_ref[...], preferred_element_type=jnp.float32)
    out_ref[...] = p_ref[0] + p_ref[1] + xr + bias_ref[...]


def _tc_combine(parts, x, root, bias2d, bn, interpret=False):
    n, nin = x.shape
    nout = root.shape[1]
    grid = n // bn
    return pl.pallas_call(
        _combine_tc,
        grid=(grid,),
        in_specs=[
            pl.BlockSpec((NC, bn, nout), lambda i: (0, i, 0)),
            pl.BlockSpec((bn, nin), lambda i: (i, 0)),
            pl.BlockSpec((nin, nout), lambda i: (0, 0)),
            pl.BlockSpec((1, nout), lambda i: (0, 0)),
        ],
        out_specs=pl.BlockSpec((bn, nout), lambda i: (i, 0)),
        out_shape=jax.ShapeDtypeStruct((n, nout), jnp.float32),
        interpret=interpret,
    )(parts, x, root, bias2d)


def kernel(x, edge_index, edge_attr, mlp_w1, mlp_w2, root, bias):
    n, nin = x.shape
    e = edge_index.shape[1]
    hid = mlp_w1.shape[1]
    nout = root.shape[1]
    assert n % NS == 0

    nch = -(-e // (NW * CH))
    e_pad = NW * nch * CH
    src = edge_index[0]
    dst = edge_index[1]
    pad = e_pad - e
    if pad:
        src = jnp.concatenate([src, jnp.zeros((pad,), jnp.int32)])
        dst = jnp.concatenate([dst, jnp.full((pad,), n, jnp.int32)])
        ea = jnp.concatenate([edge_attr,
                              jnp.zeros((pad, nin), edge_attr.dtype)])
    else:
        ea = edge_attr
    src_g = src.reshape(NW, nch, CH)
    dst_g = dst.reshape(NW, nch, CH)

    n_acc = -(-(n + 1) // NS) * NS
    zeros_acc = jnp.zeros((n_acc, nout), jnp.float32)

    ii = lax.broadcasted_iota(jnp.int32, (nin, hid), 0)
    cc = lax.broadcasted_iota(jnp.int32, (nin, hid), 1)
    rep = (cc // nout == ii).astype(jnp.float32)
    c2 = lax.broadcasted_iota(jnp.int32, (hid, nout), 0)
    oo = lax.broadcasted_iota(jnp.int32, (hid, nout), 1)
    sel = (c2 % nout == oo).astype(jnp.float32)

    xj = _sc_gather(x, src_g, nin, nch)
    msg = _tc_edge(ea, xj, mlp_w1, mlp_w2, rep, sel, be=2048)
    parts = _sc_scatter(msg, dst_g, zeros_acc, n_acc, n, nout, nch)
    out = _tc_combine(parts, x, root, bias.reshape(1, nout), bn=2000)
    return out


# speedup vs baseline: 3.3280x; 1.0024x over previous
import functools

import jax
import jax.numpy as jnp
from jax import lax
from jax.experimental import pallas as pl
from jax.experimental.pallas import tpu as pltpu
from jax.experimental.pallas import tpu_sc as plsc

NC, NS = 2, 16
NW = NC * NS
CH = 128


def _gather_body(nch, x_hbm, idx_hbm, out_hbm, idx_v, rows_v, sem):
    c = lax.axis_index("c")
    s = lax.axis_index("s")
    wid = s * NC + c
    epw = nch * CH
    pltpu.sync_copy(idx_hbm.at[wid], idx_v)

    def fire(j, carry):
        pltpu.make_async_copy(
            x_hbm.at[idx_v.at[j]],
            rows_v.at[pl.ds(j * CH, CH)],
            sem,
        ).start()
        return carry

    def drain(j, carry):
        pltpu.make_async_copy(
            x_hbm.at[idx_v.at[0]],
            rows_v.at[pl.ds(0, CH)],
            sem,
        ).wait()
        return carry

    lax.fori_loop(0, nch, fire, 0)
    lax.fori_loop(0, nch, drain, 0)
    pltpu.sync_copy(rows_v, out_hbm.at[pl.ds(wid * epw, epw)])


def _sc_gather(x, idx_grouped, nin, nch):
    epw = nch * CH
    run = pl.kernel(
        functools.partial(_gather_body, nch),
        out_type=jax.ShapeDtypeStruct((NW * epw, nin), jnp.float32),
        mesh=plsc.VectorSubcoreMesh(core_axis_name="c", subcore_axis_name="s"),
        scratch_types=[
            pltpu.VMEM((nch, CH), jnp.int32),
            pltpu.VMEM((epw, nin), jnp.float32),
            pltpu.SemaphoreType.DMA,
        ],
        compiler_params=pltpu.CompilerParams(use_tc_tiling_on_sc=False),
    )
    return run(x, idx_grouped)


def _scatter_body(nch, n_acc, n_out, nout, msg_hbm, idx_hbm, zeros_hbm,
                  part_hbm, idx_v, rows_v, tmp_v, acc_sh, sem):
    c = lax.axis_index("c")
    s = lax.axis_index("s")
    wid = s * NC + c
    epw = nch * CH
    cnt = n_acc // NS

    pltpu.sync_copy(zeros_hbm.at[pl.ds(s * cnt, cnt)], tmp_v)
    pltpu.sync_copy(tmp_v, acc_sh.at[pl.ds(s * cnt, cnt)])

    pltpu.sync_copy(idx_hbm.at[wid], idx_v)
    pltpu.sync_copy(msg_hbm.at[pl.ds(wid * epw, epw)], rows_v)
    plsc.subcore_barrier()

    def fire(j, carry):
        pltpu.async_copy(
            rows_v.at[pl.ds(j * CH, CH)],
            acc_sh.at[idx_v.at[j]],
            sem,
            add=True,
        )
        return carry

    def drain(j, carry):
        pltpu.make_async_copy(
            rows_v.at[pl.ds(0, CH)],
            acc_sh.at[idx_v.at[0]],
            sem,
        ).wait()
        return carry

    lax.fori_loop(0, nch, fire, 0)
    lax.fori_loop(0, nch, drain, 0)
    plsc.subcore_barrier()

    ocnt = n_out // NS
    pltpu.sync_copy(acc_sh.at[pl.ds(s * ocnt, ocnt)], tmp_v.at[pl.ds(0, ocnt)])
    pltpu.sync_copy(tmp_v.at[pl.ds(0, ocnt)],
                    part_hbm.at[c].at[pl.ds(s * ocnt, ocnt)])


def _sc_scatter(msg, idx_grouped, zeros_acc, n_acc, n_out, nout, nch):
    epw = nch * CH
    run = pl.kernel(
        functools.partial(_scatter_body, nch, n_acc, n_out, nout),
        out_type=jax.ShapeDtypeStruct((NC, n_out, nout), jnp.float32),
        mesh=plsc.VectorSubcoreMesh(core_axis_name="c", subcore_axis_name="s"),
        scratch_types=[
            pltpu.VMEM((nch, CH), jnp.int32),
            pltpu.VMEM((epw, nout), jnp.float32),
            pltpu.VMEM((n_acc // NS, nout), jnp.float32),
            pltpu.VMEM_SHARED((n_acc, nout), jnp.float32),
            pltpu.SemaphoreType.DMA,
        ],
        compiler_params=pltpu.CompilerParams(use_tc_tiling_on_sc=False),
    )
    return run(msg, idx_grouped, zeros_acc)


def _edge_tc(ea_ref, xj_ref, w1_ref, w2_ref, rep_ref, sel_ref, msg_ref):
    a = jnp.dot(ea_ref[...], w1_ref[...], preferred_element_type=jnp.float32)
    r = jnp.maximum(a, 0.0)
    h = jnp.dot(r.astype(jnp.bfloat16), w2_ref[...].astype(jnp.bfloat16),
                preferred_element_type=jnp.float32)
    xr = jnp.dot(xj_ref[...], rep_ref[...], preferred_element_type=jnp.float32)
    msg_ref[...] = jnp.dot(xr * h, sel_ref[...],
                           preferred_element_type=jnp.float32)


def _tc_edge(ea, xj, w1, w2, rep, sel, be, interpret=False):
    e_pad, nin = ea.shape
    hid = w2.shape[1]
    nout = sel.shape[1]
    grid = e_pad // be
    return pl.pallas_call(
        _edge_tc,
        grid=(grid,),
        in_specs=[
            pl.BlockSpec((be, nin), lambda i: (i, 0)),
            pl.BlockSpec((be, nin), lambda i: (i, 0)),
            pl.BlockSpec((nin, hid), lambda i: (0, 0)),
            pl.BlockSpec((hid, hid), lambda i: (0, 0)),
            pl.BlockSpec((nin, hid), lambda i: (0, 0)),
            pl.BlockSpec((hid, nout), lambda i: (0, 0)),
        ],
        out_specs=pl.BlockSpec((be, nout), lambda i: (i, 0)),
        out_shape=jax.ShapeDtypeStruct((e_pad, nout), jnp.float32),
        interpret=interpret,
    )(ea, xj, w1, w2, rep, sel)


def _combine_tc(p_ref, x_ref, root_ref, bias_ref, out_ref):
    xr = jnp.dot(x_ref[...], root_ref[...], preferred_element_type=jnp.float32)
    out_ref[...] = p_ref[0] + p_ref[1] + xr + bias_ref[...]


def _tc_combine(parts, x, root, bias2d, bn, interpret=False):
    n, nin = x.shape
    nout = root.shape[1]
    grid = n // bn
    return pl.pallas_call(
        _combine_tc,
        grid=(grid,),
        in_specs=[
            pl.BlockSpec((NC, bn, nout), lambda i: (0, i, 0)),
            pl.BlockSpec((bn, nin), lambda i: (i, 0)),
            pl.BlockSpec((nin, nout), lambda i: (0, 0)),
            pl.BlockSpec((1, nout), lambda i: (0, 0)),
        ],
        out_specs=pl.BlockSpec((bn, nout), lambda i: (i, 0)),
        out_shape=jax.ShapeDtypeStruct((n, nout), jnp.float32),
        interpret=interpret,
    )(parts, x, root, bias2d)


def kernel(x, edge_index, edge_attr, mlp_w1, mlp_w2, root, bias):
    n, nin = x.shape
    e = edge_index.shape[1]
    hid = mlp_w1.shape[1]
    nout = root.shape[1]
    assert n % NS == 0

    nch = -(-e // (NW * CH))
    e_pad = NW * nch * CH
    src = edge_index[0]
    dst = edge_index[1]
    pad = e_pad - e
    if pad:
        src = jnp.concatenate([src, jnp.zeros((pad,), jnp.int32)])
        dst = jnp.concatenate([dst, jnp.full((pad,), n, jnp.int32)])
        ea = jnp.concatenate([edge_attr,
                              jnp.zeros((pad, nin), edge_attr.dtype)])
    else:
        ea = edge_attr
    src_g = src.reshape(NW, nch, CH)
    dst_g = dst.reshape(NW, nch, CH)

    n_acc = -(-(n + 1) // NS) * NS
    zeros_acc = jnp.zeros((n_acc, nout), jnp.float32)

    ii = lax.broadcasted_iota(jnp.int32, (nin, hid), 0)
    cc = lax.broadcasted_iota(jnp.int32, (nin, hid), 1)
    rep = (cc // nout == ii).astype(jnp.float32)
    c2 = lax.broadcasted_iota(jnp.int32, (hid, nout), 0)
    oo = lax.broadcasted_iota(jnp.int32, (hid, nout), 1)
    sel = (c2 % nout == oo).astype(jnp.float32)

    xj = _sc_gather(x, src_g, nin, nch)
    msg = _tc_edge(ea, xj, mlp_w1, mlp_w2, rep, sel, be=2048)
    parts = _sc_scatter(msg, dst_g, zeros_acc, n_acc, n, nout, nch)
    out = _tc_combine(parts, x, root, bias.reshape(1, nout), bn=2000)
    return out
